# baseline (speedup 1.0000x reference)
import functools

import jax
import jax.numpy as jnp
from jax import lax
from jax.experimental import pallas as pl
from jax.experimental.pallas import tpu as pltpu
from jax.experimental.pallas import tpu_sc as plsc

N2, N1 = 10000, 1000
E2, E1 = 320000, 16000
NC, NS, L = 2, 16, 16
NW = NC * NS
EPW0 = E2 // NW
EPW1 = E1 // NW
EPW1P = 512
N1P = 1024
CW = 96
CH = 105
N2P = 10112
RPT = N2P // NS

_MESH = plsc.VectorSubcoreMesh(core_axis_name="c", subcore_axis_name="s")
_SC_PARAMS = pltpu.CompilerParams(needs_layout_passes=False)


def _f32(*shape):
    return jax.ShapeDtypeStruct(shape, jnp.float32)


@functools.partial(
    pl.kernel,
    out_type=(_f32(NW, N2), _f32(NW, N1P)),
    mesh=_MESH,
    compiler_params=_SC_PARAMS,
    scratch_types=[
        pltpu.VMEM((EPW0,), jnp.int32),
        pltpu.VMEM((N2,), jnp.float32),
        pltpu.VMEM((EPW1P,), jnp.int32),
        pltpu.VMEM((1024,), jnp.float32),
    ],
)
def _sc_degrees(dst0_hbm, dst1_hbm, deg0p, deg1p, idx0_v, deg0_v, idx1_v, deg1_v):
    cid = lax.axis_index("c")
    sid = lax.axis_index("s")
    wid = cid * NS + sid
    ones = jnp.ones((L,), jnp.float32)
    zeros = jnp.zeros((L,), jnp.float32)

    pltpu.sync_copy(dst0_hbm.at[wid], idx0_v)

    def zero0(i, _):
        deg0_v[pl.ds(i * L, L)] = zeros
        return 0

    lax.fori_loop(0, N2 // L, zero0, 0)

    def acc0(i, _):
        idx = idx0_v[pl.ds(i * L, L)]
        plsc.addupdate_scatter(deg0_v, [idx], ones)
        return 0

    lax.fori_loop(0, EPW0 // L, acc0, 0)
    pltpu.sync_copy(deg0_v, deg0p.at[wid])

    def zero1(i, _):
        deg1_v[pl.ds(i * L, L)] = zeros
        return 0

    lax.fori_loop(0, 1024 // L, zero1, 0)
    pltpu.sync_copy(dst1_hbm.at[wid], idx1_v)

    def acc1(i, _):
        idx = idx1_v[pl.ds(i * L, L)]
        plsc.addupdate_scatter(deg1_v, [idx], ones)
        return 0

    lax.fori_loop(0, EPW1P // L, acc1, 0)
    pltpu.sync_copy(deg1_v, deg1p.at[wid])


@functools.partial(
    pl.kernel,
    out_type=_f32(NW, N1P),
    mesh=_MESH,
    compiler_params=_SC_PARAMS,
    scratch_types=[
        pltpu.VMEM((N1P,), jnp.float32),
        pltpu.VMEM((N1P,), jnp.float32),
        pltpu.VMEM((EPW1P,), jnp.int32),
        pltpu.VMEM((EPW1P,), jnp.int32),
        pltpu.VMEM((N1P,), jnp.float32),
    ],
)
def _sc_a1_agg(s1_hbm, dis1_hbm, src1_hbm, dst1_hbm, t1p,
               s1_v, dis1_v, src_v, dst_v, t1_v):
    cid = lax.axis_index("c")
    sid = lax.axis_index("s")
    wid = cid * NS + sid

    pltpu.sync_copy(s1_hbm, s1_v)
    pltpu.sync_copy(dis1_hbm, dis1_v)

    def zero(i, _):
        t1_v[pl.ds(i * L, L)] = jnp.zeros((L,), jnp.float32)
        return 0

    lax.fori_loop(0, 1024 // L, zero, 0)

    pltpu.sync_copy(src1_hbm.at[wid], src_v)
    pltpu.sync_copy(dst1_hbm.at[wid], dst_v)

    def acc(i, _):
        sidx = src_v[pl.ds(i * L, L)]
        didx = dst_v[pl.ds(i * L, L)]
        vals = plsc.load_gather(s1_v, [sidx]) * plsc.load_gather(dis1_v, [sidx])
        plsc.addupdate_scatter(t1_v, [didx], vals)
        return 0

    lax.fori_loop(0, EPW1P // L, acc, 0)
    pltpu.sync_copy(t1_v, t1p.at[wid])


def _agg_pass(q_hbm, src_v, dst_v, rows_v, acc_sh, sems, nch):
    def step(j, _):
        pltpu.async_copy(q_hbm.at[src_v.at[j]], rows_v, sems).wait()
        pltpu.sync_copy(rows_v, acc_sh.at[dst_v.at[j]], add=True)
        return 0

    lax.fori_loop(0, nch, step, 0)


_AGG_SCRATCH = [
    pltpu.VMEM((CH, CW), jnp.int32),
    pltpu.VMEM((CH, CW), jnp.int32),
    pltpu.VMEM((CW, 128), jnp.float32),
    pltpu.VMEM_SHARED((N2P, 128), jnp.float32),
    pltpu.SemaphoreType.DMA,
]


@functools.partial(
    pl.kernel,
    out_type=_f32(NC, N2P, 128),
    mesh=_MESH,
    compiler_params=_SC_PARAMS,
    scratch_types=_AGG_SCRATCH,
)
def _sc_agg128(q_hbm, src_hbm, dst_hbm, zeros_hbm, outp,
               src_v, dst_v, rows_v, acc_sh, sems):
    cid = lax.axis_index("c")
    sid = lax.axis_index("s")
    wid = cid * NS + sid
    base = sid * RPT

    pltpu.sync_copy(src_hbm.at[wid], src_v)
    pltpu.sync_copy(dst_hbm.at[wid], dst_v)
    pltpu.sync_copy(zeros_hbm, acc_sh.at[pl.ds(base, RPT)])
    plsc.subcore_barrier()
    _agg_pass(q_hbm, src_v, dst_v, rows_v, acc_sh, sems, CH)
    plsc.subcore_barrier()
    pltpu.sync_copy(acc_sh.at[pl.ds(base, RPT)],
                    outp.at[cid].at[pl.ds(base, RPT)])


def _tc_a_body(deg0p, deg1p, z, wd, bd, delta, u1, wg0,
               dis0_o, dis1_o, s1_o, w1_o, vl_o):
    deg0 = jnp.sum(deg0p[...], axis=0, keepdims=True) + 1.0
    dis0_o[...] = lax.rsqrt(deg0)
    deg1 = jnp.sum(deg1p[...], axis=0, keepdims=True) + 1.0
    dis1_o[...] = lax.rsqrt(deg1)
    s1_o[...] = jnp.sum(u1[...], axis=1, keepdims=True)
    x = jnp.dot(z[...], wd[...], preferred_element_type=jnp.float32) \
        + bd[...] + delta[...]
    w1_o[...] = jnp.dot(x[:, :128], wg0[...], preferred_element_type=jnp.float32)
    vl_o[...] = x[:, 128:]


def _tc_bg_body(t1pt, dis1c, s1c, w1, bg0, wg1, u0, dis0c, q128_o):
    q1c = dis1c[...] * s1c[...]
    t1c = jnp.sum(t1pt[...], axis=1, keepdims=True)
    a1 = dis1c[...] * (t1c + q1c)
    pre = a1 * w1[...] + bg0[...]
    xg1 = jnp.where(pre >= 0, pre, 0.01 * pre)
    g = jnp.dot(xg1, wg1[...], preferred_element_type=jnp.float32)
    h2 = jnp.dot(u0[...], g, preferred_element_type=jnp.float32)
    q128_o[...] = h2 * dis0c[...]


def _tc_wl_body(vl, wl, bl, out_o):
    out_o[...] = jnp.dot(vl[...], wl[...], preferred_element_type=jnp.float32) \
        + bl[...]


def _tc_bl_body(xl0, dis0c, wc0, wp0, bp0, q64_o, y1p_o):
    h0 = jnp.dot(xl0[...], wc0[...], preferred_element_type=jnp.float32)
    q64_o[...] = h0 * dis0c[...]
    y1p_o[...] = jnp.dot(xl0[...], wp0[...], preferred_element_type=jnp.float32) \
        + bp0[...]


def _tc_c_body(tpa, tpb, q128, q64, dis0c, y1p, bg1, bc0, wc1,
               xg_o, y1_o, q2_o):
    d = dis0c[...]
    ta = tpa[0] + tpa[1]
    xg_o[...] = d * (ta + q128[...]) + bg1[...]
    tb = tpb[0, :, :64] + tpb[1, :, :64]
    c0 = d * (tb + q64[...]) + bc0[...]
    y1 = jnp.where(c0 >= 0, c0, 0.01 * c0) + y1p[...]
    y1_o[...] = y1
    q2_o[...] = jnp.dot(y1, wc1[...], preferred_element_type=jnp.float32) * d


def _tc_d_body(t2p, q2, dis0c, y1, xg, bc1, wp1, bp1, wa0g, wa0l, ba0, wa1,
               ba1, out_o):
    t2 = t2p[0] + t2p[1]
    xl = dis0c[...] * (t2 + q2[...]) + bc1[...] \
        + jnp.dot(y1[...], wp1[...], preferred_element_type=jnp.float32) \
        + bp1[...]
    xgv = xg[...]
    h = jnp.dot(xgv, wa0g[...], preferred_element_type=jnp.float32) \
        + jnp.dot(xl, wa0l[...], preferred_element_type=jnp.float32) + ba0[...]
    h = jnp.maximum(h, 0.0)
    la = jnp.dot(h, wa1[...], preferred_element_type=jnp.float32) + ba1[...]
    d = la[:, 0:1] - la[:, 1:2]
    w0 = 1.0 / (1.0 + jnp.exp(-d))
    out_o[...] = w0 * xgv + (1.0 - w0) * xl


_RB = 1000
_NRB = N2 // _RB


def _row_spec(w):
    return pl.BlockSpec((_RB, w), lambda i: (i, 0))


def _full_spec(shape):
    return pl.BlockSpec(shape, lambda i: tuple(0 for _ in shape))


def kernel(z, Wd, bd, Wl, bl, Wg0, bg0, Wg1, bg1, Wc0, bc0, Wc1, bc1,
           Wp0, bp0, Wp1, bp1, Wa0, ba0, Wa1, ba1, U0, U1, A0, A1, batch_size):
    f32 = jnp.float32
    delta = (jnp.asarray(batch_size) - z.shape[0]).astype(f32).reshape(1, 1)

    dst0w = A0[1].reshape(NW, EPW0)
    pad = ((0, 0), (0, EPW1P - EPW1))
    src1w = jnp.pad(A1[0].reshape(NW, EPW1), pad)
    dst1w = jnp.pad(A1[1].reshape(NW, EPW1), pad,
                    constant_values=N1)
    npad = CH * CW - EPW0
    junk = N2 + jnp.arange(npad, dtype=jnp.int32) % (N2P - N2)
    src0t = jnp.pad(A0[0].reshape(NW, EPW0),
                    ((0, 0), (0, npad))).reshape(NW, CH, CW)
    dst0t = jnp.concatenate(
        [A0[1].reshape(NW, EPW0), jnp.broadcast_to(junk, (NW, npad))],
        axis=1).reshape(NW, CH, CW)

    deg0p, deg1p = _sc_degrees(dst0w, dst1w)

    dis0, dis1, s1c, w1, vl = pl.pallas_call(
        _tc_a_body,
        out_shape=[_f32(1, N2), _f32(1, N1P), _f32(N1, 1), _f32(1, 128),
                   _f32(1, 128)],
    )(deg0p, deg1p, z, Wd, bd.reshape(1, -1), delta, U1, Wg0)

    s1p = jnp.pad(s1c.reshape(N1), (0, N1P - N1))
    t1p = _sc_a1_agg(s1p, dis1.reshape(N1P), src1w, dst1w)

    dis0c = dis0.reshape(N2, 1)

    q128 = pl.pallas_call(
        _tc_bg_body,
        grid=(_NRB,),
        in_specs=[_full_spec((N1, NW)), _full_spec((N1, 1)),
                  _full_spec((N1, 1)), _full_spec((1, 128)),
                  _full_spec((1, 128)), _full_spec((128, 128)),
                  pl.BlockSpec((_RB, N1), lambda i: (i, 0)), _row_spec(1)],
        out_specs=_row_spec(128),
        out_shape=_f32(N2, 128),
    )(t1p[:, :N1].T, dis1[:, :N1].reshape(N1, 1), s1c, w1,
      bg0.reshape(1, -1), Wg1, U0, dis0c)

    xl0 = pl.pallas_call(
        _tc_wl_body,
        grid=(10,),
        in_specs=[_full_spec((1, 128)),
                  pl.BlockSpec((128, 16000), lambda i: (0, i)),
                  pl.BlockSpec((1, 16000), lambda i: (0, i))],
        out_specs=pl.BlockSpec((1, 16000), lambda i: (0, i)),
        out_shape=_f32(1, 16 * N2),
    )(vl, Wl, bl.reshape(1, -1)).reshape(N2, 16)

    q64, y1p = pl.pallas_call(
        _tc_bl_body,
        grid=(_NRB,),
        in_specs=[_row_spec(16), _row_spec(1),
                  _full_spec((16, 64)), _full_spec((16, 64)),
                  _full_spec((1, 64))],
        out_specs=[_row_spec(64), _row_spec(64)],
        out_shape=[_f32(N2, 64), _f32(N2, 64)],
    )(xl0, dis0c, Wc0, Wp0, bp0.reshape(1, -1))

    zeros128 = jnp.zeros((RPT, 128), f32)
    qb = jnp.pad(q64, ((0, 0), (0, 64)))
    tpa = _sc_agg128(q128, src0t, dst0t, zeros128)
    tpb = _sc_agg128(qb, src0t, dst0t, zeros128)

    xg, y1, q2 = pl.pallas_call(
        _tc_c_body,
        grid=(_NRB,),
        in_specs=[pl.BlockSpec((NC, _RB, 128), lambda i: (0, i, 0)),
                  pl.BlockSpec((NC, _RB, 128), lambda i: (0, i, 0)),
                  _row_spec(128), _row_spec(64), _row_spec(1), _row_spec(64),
                  _full_spec((1, 128)), _full_spec((1, 64)),
                  _full_spec((64, 128))],
        out_specs=[_row_spec(128), _row_spec(64), _row_spec(128)],
        out_shape=[_f32(N2, 128), _f32(N2, 64), _f32(N2, 128)],
    )(tpa, tpb, q128, q64, dis0c, y1p, bg1.reshape(1, -1),
      bc0.reshape(1, -1), Wc1)

    t2p = _sc_agg128(q2, src0t, dst0t, zeros128)

    out = pl.pallas_call(
        _tc_d_body,
        grid=(_NRB,),
        in_specs=[pl.BlockSpec((NC, _RB, 128), lambda i: (0, i, 0)),
                  _row_spec(128), _row_spec(1), _row_spec(64), _row_spec(128),
                  _full_spec((1, 128)), _full_spec((64, 128)),
                  _full_spec((1, 128)), _full_spec((128, 64)),
                  _full_spec((128, 64)), _full_spec((1, 64)),
                  _full_spec((64, 2)), _full_spec((1, 2))],
        out_specs=_row_spec(128),
        out_shape=_f32(N2, 128),
    )(t2p, q2, dis0c, y1, xg, bc1.reshape(1, -1), Wp1, bp1.reshape(1, -1),
      Wa0[:128], Wa0[128:], ba0.reshape(1, -1), Wa1, ba1.reshape(1, -1))

    return out


# speedup vs baseline: 1.2808x; 1.2808x over previous
import functools

import jax
import jax.numpy as jnp
from jax import lax
from jax.experimental import pallas as pl
from jax.experimental.pallas import tpu as pltpu
from jax.experimental.pallas import tpu_sc as plsc

N2, N1 = 10000, 1000
E2, E1 = 320000, 16000
NC, NS, L = 2, 16, 16
NW = NC * NS
EPW0 = E2 // NW
EPW1 = E1 // NW
EPW1P = 512
N1P = 1024
CW = 80
CH = 125
N2P = 10112
RPT = N2P // NS

_MESH = plsc.VectorSubcoreMesh(core_axis_name="c", subcore_axis_name="s")
_SC_PARAMS = pltpu.CompilerParams(needs_layout_passes=False)


def _f32(*shape):
    return jax.ShapeDtypeStruct(shape, jnp.float32)


@functools.partial(
    pl.kernel,
    out_type=(_f32(NW, N2), _f32(NW, N1P)),
    mesh=_MESH,
    compiler_params=_SC_PARAMS,
    scratch_types=[
        pltpu.VMEM((EPW0,), jnp.int32),
        pltpu.VMEM((N2,), jnp.float32),
        pltpu.VMEM((EPW1P,), jnp.int32),
        pltpu.VMEM((1024,), jnp.float32),
    ],
)
def _sc_degrees(dst0_hbm, dst1_hbm, deg0p, deg1p, idx0_v, deg0_v, idx1_v, deg1_v):
    cid = lax.axis_index("c")
    sid = lax.axis_index("s")
    wid = cid * NS + sid
    ones = jnp.ones((L,), jnp.float32)
    zeros = jnp.zeros((L,), jnp.float32)

    pltpu.sync_copy(dst0_hbm.at[wid], idx0_v)

    def zero0(i, _):
        deg0_v[pl.ds(i * L, L)] = zeros
        return 0

    lax.fori_loop(0, N2 // L, zero0, 0)

    def acc0(i, _):
        idx = idx0_v[pl.ds(i * L, L)]
        plsc.addupdate_scatter(deg0_v, [idx], ones)
        return 0

    lax.fori_loop(0, EPW0 // L, acc0, 0)
    pltpu.sync_copy(deg0_v, deg0p.at[wid])

    def zero1(i, _):
        deg1_v[pl.ds(i * L, L)] = zeros
        return 0

    lax.fori_loop(0, 1024 // L, zero1, 0)
    pltpu.sync_copy(dst1_hbm.at[wid], idx1_v)

    def acc1(i, _):
        idx = idx1_v[pl.ds(i * L, L)]
        plsc.addupdate_scatter(deg1_v, [idx], ones)
        return 0

    lax.fori_loop(0, EPW1P // L, acc1, 0)
    pltpu.sync_copy(deg1_v, deg1p.at[wid])


@functools.partial(
    pl.kernel,
    out_type=_f32(NW, N1P),
    mesh=_MESH,
    compiler_params=_SC_PARAMS,
    scratch_types=[
        pltpu.VMEM((N1P,), jnp.float32),
        pltpu.VMEM((N1P,), jnp.float32),
        pltpu.VMEM((EPW1P,), jnp.int32),
        pltpu.VMEM((EPW1P,), jnp.int32),
        pltpu.VMEM((N1P,), jnp.float32),
    ],
)
def _sc_a1_agg(s1_hbm, dis1_hbm, src1_hbm, dst1_hbm, t1p,
               s1_v, dis1_v, src_v, dst_v, t1_v):
    cid = lax.axis_index("c")
    sid = lax.axis_index("s")
    wid = cid * NS + sid

    pltpu.sync_copy(s1_hbm, s1_v)
    pltpu.sync_copy(dis1_hbm, dis1_v)

    def zero(i, _):
        t1_v[pl.ds(i * L, L)] = jnp.zeros((L,), jnp.float32)
        return 0

    lax.fori_loop(0, 1024 // L, zero, 0)

    pltpu.sync_copy(src1_hbm.at[wid], src_v)
    pltpu.sync_copy(dst1_hbm.at[wid], dst_v)

    def acc(i, _):
        sidx = src_v[pl.ds(i * L, L)]
        didx = dst_v[pl.ds(i * L, L)]
        vals = plsc.load_gather(s1_v, [sidx]) * plsc.load_gather(dis1_v, [sidx])
        plsc.addupdate_scatter(t1_v, [didx], vals)
        return 0

    lax.fori_loop(0, EPW1P // L, acc, 0)
    pltpu.sync_copy(t1_v, t1p.at[wid])


def _agg_pass(q_hbm, src_v, dst_v, rows_v, acc_sh, sems, nch):
    def step(j, _):
        pltpu.async_copy(q_hbm.at[src_v.at[j]], rows_v, sems).wait()
        pltpu.sync_copy(rows_v, acc_sh.at[dst_v.at[j]], add=True)
        return 0

    lax.fori_loop(0, nch, step, 0)


_AGG_SCRATCH = [
    pltpu.VMEM((CH, CW), jnp.int32),
    pltpu.VMEM((CH, CW), jnp.int32),
    pltpu.VMEM((CW, 128), jnp.float32),
    pltpu.VMEM_SHARED((N2P, 128), jnp.float32),
    pltpu.SemaphoreType.DMA,
]


@functools.partial(
    pl.kernel,
    out_type=_f32(NC, N2P, 128),
    mesh=_MESH,
    compiler_params=_SC_PARAMS,
    scratch_types=_AGG_SCRATCH,
)
def _sc_agg128(q_hbm, src_hbm, dst_hbm, zeros_hbm, outp,
               src_v, dst_v, rows_v, acc_sh, sems):
    cid = lax.axis_index("c")
    sid = lax.axis_index("s")
    wid = cid * NS + sid
    base = sid * RPT

    pltpu.sync_copy(src_hbm.at[wid], src_v)
    pltpu.sync_copy(dst_hbm.at[wid], dst_v)
    pltpu.sync_copy(zeros_hbm, acc_sh.at[pl.ds(base, RPT)])
    plsc.subcore_barrier()
    _agg_pass(q_hbm, src_v, dst_v, rows_v, acc_sh, sems, CH)
    plsc.subcore_barrier()
    pltpu.sync_copy(acc_sh.at[pl.ds(base, RPT)],
                    outp.at[cid].at[pl.ds(base, RPT)])


def _tc_a_body(deg0p, deg1p, z, wd, bd, delta, u1, wg0,
               dis0_o, dis1_o, s1_o, w1_o, vl_o):
    deg0 = jnp.sum(deg0p[...], axis=0, keepdims=True) + 1.0
    dis0_o[...] = lax.rsqrt(deg0)
    deg1 = jnp.sum(deg1p[...], axis=0, keepdims=True) + 1.0
    dis1_o[...] = lax.rsqrt(deg1)
    s1_o[...] = jnp.sum(u1[...], axis=1, keepdims=True)
    x = jnp.dot(z[...], wd[...], preferred_element_type=jnp.float32) \
        + bd[...] + delta[...]
    w1_o[...] = jnp.dot(x[:, :128], wg0[...], preferred_element_type=jnp.float32)
    vl_o[...] = x[:, 128:]


def _tc_bg_body(t1pt, dis1c, s1c, w1, bg0, wg1, u0, dis0c, q128_o):
    q1c = dis1c[...] * s1c[...]
    t1c = jnp.sum(t1pt[...], axis=1, keepdims=True)
    a1 = dis1c[...] * (t1c + q1c)
    pre = a1 * w1[...] + bg0[...]
    xg1 = jnp.where(pre >= 0, pre, 0.01 * pre)
    g = jnp.dot(xg1, wg1[...], preferred_element_type=jnp.float32)
    h2 = jnp.dot(u0[...], g, preferred_element_type=jnp.float32)
    q128_o[...] = h2 * dis0c[...]


def _tc_wl_body(vl, wl, bl, out_o):
    out_o[...] = jnp.dot(vl[...], wl[...], preferred_element_type=jnp.float32) \
        + bl[...]


def _tc_bl_body(xl0, dis0c, wc0, wp0, bp0, q64_o, y1p_o):
    h0 = jnp.dot(xl0[...], wc0[...], preferred_element_type=jnp.float32)
    q64_o[...] = h0 * dis0c[...]
    y1p_o[...] = jnp.dot(xl0[...], wp0[...], preferred_element_type=jnp.float32) \
        + bp0[...]


def _tc_c_body(tpa, tpb, q128, q64, dis0c, y1p, bg1, bc0, wc1,
               xg_o, y1_o, q2_o):
    d = dis0c[...]
    ta = tpa[0] + tpa[1]
    xg_o[...] = d * (ta + q128[...]) + bg1[...]
    tb = tpb[0, :, :64] + tpb[1, :, :64]
    c0 = d * (tb + q64[...]) + bc0[...]
    y1 = jnp.where(c0 >= 0, c0, 0.01 * c0) + y1p[...]
    y1_o[...] = y1
    q2_o[...] = jnp.dot(y1, wc1[...], preferred_element_type=jnp.float32) * d


def _tc_d_body(t2p, q2, dis0c, y1, xg, bc1, wp1, bp1, wa0g, wa0l, ba0, wa1,
               ba1, out_o):
    t2 = t2p[0] + t2p[1]
    xl = dis0c[...] * (t2 + q2[...]) + bc1[...] \
        + jnp.dot(y1[...], wp1[...], preferred_element_type=jnp.float32) \
        + bp1[...]
    xgv = xg[...]
    h = jnp.dot(xgv, wa0g[...], preferred_element_type=jnp.float32) \
        + jnp.dot(xl, wa0l[...], preferred_element_type=jnp.float32) + ba0[...]
    h = jnp.maximum(h, 0.0)
    la = jnp.dot(h, wa1[...], preferred_element_type=jnp.float32) + ba1[...]
    d = la[:, 0:1] - la[:, 1:2]
    w0 = 1.0 / (1.0 + jnp.exp(-d))
    out_o[...] = w0 * xgv + (1.0 - w0) * xl


_RB = 1000
_NRB = N2 // _RB


def _row_spec(w):
    return pl.BlockSpec((_RB, w), lambda i: (i, 0))


def _full_spec(shape):
    return pl.BlockSpec(shape, lambda i: tuple(0 for _ in shape))


def kernel(z, Wd, bd, Wl, bl, Wg0, bg0, Wg1, bg1, Wc0, bc0, Wc1, bc1,
           Wp0, bp0, Wp1, bp1, Wa0, ba0, Wa1, ba1, U0, U1, A0, A1, batch_size):
    f32 = jnp.float32
    delta = (jnp.asarray(batch_size) - z.shape[0]).astype(f32).reshape(1, 1)

    dst0w = A0[1].reshape(NW, EPW0)
    pad = ((0, 0), (0, EPW1P - EPW1))
    src1w = jnp.pad(A1[0].reshape(NW, EPW1), pad)
    dst1w = jnp.pad(A1[1].reshape(NW, EPW1), pad,
                    constant_values=N1)
    src0t = A0[0].reshape(NW, CH, CW)
    dst0t = A0[1].reshape(NW, CH, CW)

    deg0p, deg1p = _sc_degrees(dst0w, dst1w)

    dis0, dis1, s1c, w1, vl = pl.pallas_call(
        _tc_a_body,
        out_shape=[_f32(1, N2), _f32(1, N1P), _f32(N1, 1), _f32(1, 128),
                   _f32(1, 128)],
    )(deg0p, deg1p, z, Wd, bd.reshape(1, -1), delta, U1, Wg0)

    s1p = jnp.pad(s1c.reshape(N1), (0, N1P - N1))
    t1p = _sc_a1_agg(s1p, dis1.reshape(N1P), src1w, dst1w)

    dis0c = dis0.reshape(N2, 1)

    q128 = pl.pallas_call(
        _tc_bg_body,
        grid=(_NRB,),
        in_specs=[_full_spec((N1, NW)), _full_spec((N1, 1)),
                  _full_spec((N1, 1)), _full_spec((1, 128)),
                  _full_spec((1, 128)), _full_spec((128, 128)),
                  pl.BlockSpec((_RB, N1), lambda i: (i, 0)), _row_spec(1)],
        out_specs=_row_spec(128),
        out_shape=_f32(N2, 128),
    )(t1p[:, :N1].T, dis1[:, :N1].reshape(N1, 1), s1c, w1,
      bg0.reshape(1, -1), Wg1, U0, dis0c)

    xl0 = pl.pallas_call(
        _tc_wl_body,
        grid=(10,),
        in_specs=[_full_spec((1, 128)),
                  pl.BlockSpec((128, 16000), lambda i: (0, i)),
                  pl.BlockSpec((1, 16000), lambda i: (0, i))],
        out_specs=pl.BlockSpec((1, 16000), lambda i: (0, i)),
        out_shape=_f32(1, 16 * N2),
    )(vl, Wl, bl.reshape(1, -1)).reshape(N2, 16)

    q64, y1p = pl.pallas_call(
        _tc_bl_body,
        grid=(_NRB,),
        in_specs=[_row_spec(16), _row_spec(1),
                  _full_spec((16, 64)), _full_spec((16, 64)),
                  _full_spec((1, 64))],
        out_specs=[_row_spec(64), _row_spec(64)],
        out_shape=[_f32(N2, 64), _f32(N2, 64)],
    )(xl0, dis0c, Wc0, Wp0, bp0.reshape(1, -1))

    zeros128 = jnp.zeros((RPT, 128), f32)
    qb = jnp.pad(q64, ((0, 0), (0, 64)))
    tpa = _sc_agg128(q128, src0t, dst0t, zeros128)
    tpb = _sc_agg128(qb, src0t, dst0t, zeros128)

    xg, y1, q2 = pl.pallas_call(
        _tc_c_body,
        grid=(_NRB,),
        in_specs=[pl.BlockSpec((NC, _RB, 128), lambda i: (0, i, 0)),
                  pl.BlockSpec((NC, _RB, 128), lambda i: (0, i, 0)),
                  _row_spec(128), _row_spec(64), _row_spec(1), _row_spec(64),
                  _full_spec((1, 128)), _full_spec((1, 64)),
                  _full_spec((64, 128))],
        out_specs=[_row_spec(128), _row_spec(64), _row_spec(128)],
        out_shape=[_f32(N2, 128), _f32(N2, 64), _f32(N2, 128)],
    )(tpa, tpb, q128, q64, dis0c, y1p, bg1.reshape(1, -1),
      bc0.reshape(1, -1), Wc1)

    t2p = _sc_agg128(q2, src0t, dst0t, zeros128)

    out = pl.pallas_call(
        _tc_d_body,
        grid=(_NRB,),
        in_specs=[pl.BlockSpec((NC, _RB, 128), lambda i: (0, i, 0)),
                  _row_spec(128), _row_spec(1), _row_spec(64), _row_spec(128),
                  _full_spec((1, 128)), _full_spec((64, 128)),
                  _full_spec((1, 128)), _full_spec((128, 64)),
                  _full_spec((128, 64)), _full_spec((1, 64)),
                  _full_spec((64, 2)), _full_spec((1, 2))],
        out_specs=_row_spec(128),
        out_shape=_f32(N2, 128),
    )(t2p, q2, dis0c, y1, xg, bc1.reshape(1, -1), Wp1, bp1.reshape(1, -1),
      Wa0[:128], Wa0[128:], ba0.reshape(1, -1), Wa1, ba1.reshape(1, -1))

    return out


# speedup vs baseline: 1.3009x; 1.0157x over previous
import functools

import jax
import jax.numpy as jnp
from jax import lax
from jax.experimental import pallas as pl
from jax.experimental.pallas import tpu as pltpu
from jax.experimental.pallas import tpu_sc as plsc

N2, N1 = 10000, 1000
E2, E1 = 320000, 16000
NC, NS, L = 2, 16, 16
NW = NC * NS
EPW0 = E2 // NW
EPW1 = E1 // NW
EPW1P = 512
N1P = 1024
CW = 80
CH = 125
N2P = 10112
RPT = N2P // NS

_MESH = plsc.VectorSubcoreMesh(core_axis_name="c", subcore_axis_name="s")
_SC_PARAMS = pltpu.CompilerParams(needs_layout_passes=False)


def _f32(*shape):
    return jax.ShapeDtypeStruct(shape, jnp.float32)


@functools.partial(
    pl.kernel,
    out_type=(_f32(NW, N2), _f32(NW, N1P)),
    mesh=_MESH,
    compiler_params=_SC_PARAMS,
    scratch_types=[
        pltpu.VMEM((EPW0,), jnp.int32),
        pltpu.VMEM((N2,), jnp.float32),
        pltpu.VMEM((EPW1P,), jnp.int32),
        pltpu.VMEM((1024,), jnp.float32),
    ],
)
def _sc_degrees(dst0_hbm, dst1_hbm, deg0p, deg1p, idx0_v, deg0_v, idx1_v, deg1_v):
    cid = lax.axis_index("c")
    sid = lax.axis_index("s")
    wid = cid * NS + sid
    ones = jnp.ones((L,), jnp.float32)
    zeros = jnp.zeros((L,), jnp.float32)

    pltpu.sync_copy(dst0_hbm.at[wid], idx0_v)

    def zero0(i, _):
        deg0_v[pl.ds(i * L, L)] = zeros
        return 0

    lax.fori_loop(0, N2 // L, zero0, 0)

    def acc0(i, _):
        idx = idx0_v[pl.ds(i * L, L)]
        plsc.addupdate_scatter(deg0_v, [idx], ones)
        return 0

    lax.fori_loop(0, EPW0 // L, acc0, 0)
    pltpu.sync_copy(deg0_v, deg0p.at[wid])

    def zero1(i, _):
        deg1_v[pl.ds(i * L, L)] = zeros
        return 0

    lax.fori_loop(0, 1024 // L, zero1, 0)
    pltpu.sync_copy(dst1_hbm.at[wid], idx1_v)

    def acc1(i, _):
        idx = idx1_v[pl.ds(i * L, L)]
        plsc.addupdate_scatter(deg1_v, [idx], ones)
        return 0

    lax.fori_loop(0, EPW1P // L, acc1, 0)
    pltpu.sync_copy(deg1_v, deg1p.at[wid])


@functools.partial(
    pl.kernel,
    out_type=_f32(NW, N1P),
    mesh=_MESH,
    compiler_params=_SC_PARAMS,
    scratch_types=[
        pltpu.VMEM((N1P,), jnp.float32),
        pltpu.VMEM((N1P,), jnp.float32),
        pltpu.VMEM((EPW1P,), jnp.int32),
        pltpu.VMEM((EPW1P,), jnp.int32),
        pltpu.VMEM((N1P,), jnp.float32),
    ],
)
def _sc_a1_agg(s1_hbm, dis1_hbm, src1_hbm, dst1_hbm, t1p,
               s1_v, dis1_v, src_v, dst_v, t1_v):
    cid = lax.axis_index("c")
    sid = lax.axis_index("s")
    wid = cid * NS + sid

    pltpu.sync_copy(s1_hbm, s1_v)
    pltpu.sync_copy(dis1_hbm, dis1_v)

    def zero(i, _):
        t1_v[pl.ds(i * L, L)] = jnp.zeros((L,), jnp.float32)
        return 0

    lax.fori_loop(0, 1024 // L, zero, 0)

    pltpu.sync_copy(src1_hbm.at[wid], src_v)
    pltpu.sync_copy(dst1_hbm.at[wid], dst_v)

    def acc(i, _):
        sidx = src_v[pl.ds(i * L, L)]
        didx = dst_v[pl.ds(i * L, L)]
        vals = plsc.load_gather(s1_v, [sidx]) * plsc.load_gather(dis1_v, [sidx])
        plsc.addupdate_scatter(t1_v, [didx], vals)
        return 0

    lax.fori_loop(0, EPW1P // L, acc, 0)
    pltpu.sync_copy(t1_v, t1p.at[wid])


def _agg_pass(q_hbm, src_v, dst_v, rows_v, acc_sh, sems, nch):
    def step(j, _):
        pltpu.async_copy(q_hbm.at[src_v.at[j]], rows_v, sems).wait()
        pltpu.sync_copy(rows_v, acc_sh.at[dst_v.at[j]], add=True)
        return 0

    lax.fori_loop(0, nch, step, 0)


_AGG_SCRATCH = [
    pltpu.VMEM((CH, CW), jnp.int32),
    pltpu.VMEM((CH, CW), jnp.int32),
    pltpu.VMEM((CW, 128), jnp.float32),
    pltpu.VMEM_SHARED((N2P, 128), jnp.float32),
    pltpu.SemaphoreType.DMA,
]


@functools.partial(
    pl.kernel,
    out_type=_f32(NC, N2P, 128),
    mesh=_MESH,
    compiler_params=_SC_PARAMS,
    scratch_types=_AGG_SCRATCH,
)
def _sc_agg128(q_hbm, src_hbm, dst_hbm, zeros_hbm, outp,
               src_v, dst_v, rows_v, acc_sh, sems):
    cid = lax.axis_index("c")
    sid = lax.axis_index("s")
    wid = cid * NS + sid
    base = sid * RPT

    pltpu.sync_copy(src_hbm.at[wid], src_v)
    pltpu.sync_copy(dst_hbm.at[wid], dst_v)
    pltpu.sync_copy(zeros_hbm, acc_sh.at[pl.ds(base, RPT)])
    plsc.subcore_barrier()
    _agg_pass(q_hbm, src_v, dst_v, rows_v, acc_sh, sems, CH)
    plsc.subcore_barrier()
    pltpu.sync_copy(acc_sh.at[pl.ds(base, RPT)],
                    outp.at[cid].at[pl.ds(base, RPT)])


def _tc_a_body(deg0p, deg1p, z, wd, bd, delta, u1, wg0,
               dis0_o, dis1_o, s1_o, w1_o, vl_o):
    deg0 = jnp.sum(deg0p[...], axis=0, keepdims=True) + 1.0
    dis0_o[...] = lax.rsqrt(deg0)
    deg1 = jnp.sum(deg1p[...], axis=0, keepdims=True) + 1.0
    dis1_o[...] = lax.rsqrt(deg1)
    s1_o[...] = jnp.sum(u1[...], axis=1, keepdims=True)
    x = jnp.dot(z[...], wd[...], preferred_element_type=jnp.float32) \
        + bd[...] + delta[...]
    w1_o[...] = jnp.dot(x[:, :128], wg0[...], preferred_element_type=jnp.float32)
    vl_o[...] = x[:, 128:]


def _tc_bg_body(t1pt, dis1c, s1c, w1, bg0, wg1, u0, dis0c, q128_o):
    q1c = dis1c[...] * s1c[...]
    t1c = jnp.sum(t1pt[...], axis=1, keepdims=True)
    a1 = dis1c[...] * (t1c + q1c)
    pre = a1 * w1[...] + bg0[...]
    xg1 = jnp.where(pre >= 0, pre, 0.01 * pre)
    g = jnp.dot(xg1, wg1[...], preferred_element_type=jnp.float32)
    h2 = jnp.dot(u0[...], g, preferred_element_type=jnp.float32)
    q128_o[...] = h2 * dis0c[...]


def _tc_wl_body(vl, wl, bl, out_o):
    out_o[...] = jnp.dot(vl[...], wl[...], preferred_element_type=jnp.float32) \
        + bl[...]


def _tc_bl_body(xl0, dis0c, wc0, wp0, bp0, q64_o, y1p_o):
    h0 = jnp.dot(xl0[...], wc0[...], preferred_element_type=jnp.float32)
    q64_o[...] = h0 * dis0c[...]
    y1p_o[...] = jnp.dot(xl0[...], wp0[...], preferred_element_type=jnp.float32) \
        + bp0[...]


def _tc_c_body(tpb, q64, dis0c, y1p, bc0, wc1, y1_o, q2_o):
    d = dis0c[...]
    tb = tpb[0, :, :64] + tpb[1, :, :64]
    c0 = d * (tb + q64[...]) + bc0[...]
    y1 = jnp.where(c0 >= 0, c0, 0.01 * c0) + y1p[...]
    y1_o[...] = y1
    q2_o[...] = jnp.dot(y1, wc1[...], preferred_element_type=jnp.float32) * d


def _tc_d_body(t2p, q2, dis0c, y1, tpa, q128, bg1, bc1, wp1, bp1, wa0g,
               wa0l, ba0, wa1, ba1, out_o):
    d = dis0c[...]
    t2 = t2p[0] + t2p[1]
    xl = d * (t2 + q2[...]) + bc1[...] \
        + jnp.dot(y1[...], wp1[...], preferred_element_type=jnp.float32) \
        + bp1[...]
    ta = tpa[0] + tpa[1]
    xgv = d * (ta + q128[...]) + bg1[...]
    h = jnp.dot(xgv, wa0g[...], preferred_element_type=jnp.float32) \
        + jnp.dot(xl, wa0l[...], preferred_element_type=jnp.float32) + ba0[...]
    h = jnp.maximum(h, 0.0)
    la = jnp.dot(h, wa1[...], preferred_element_type=jnp.float32) + ba1[...]
    d = la[:, 0:1] - la[:, 1:2]
    w0 = 1.0 / (1.0 + jnp.exp(-d))
    out_o[...] = w0 * xgv + (1.0 - w0) * xl


_RB = 1000
_NRB = N2 // _RB


def _row_spec(w):
    return pl.BlockSpec((_RB, w), lambda i: (i, 0))


def _full_spec(shape):
    return pl.BlockSpec(shape, lambda i: tuple(0 for _ in shape))


def kernel(z, Wd, bd, Wl, bl, Wg0, bg0, Wg1, bg1, Wc0, bc0, Wc1, bc1,
           Wp0, bp0, Wp1, bp1, Wa0, ba0, Wa1, ba1, U0, U1, A0, A1, batch_size):
    f32 = jnp.float32
    delta = (jnp.asarray(batch_size) - z.shape[0]).astype(f32).reshape(1, 1)

    dst0w = A0[1].reshape(NW, EPW0)
    pad = ((0, 0), (0, EPW1P - EPW1))
    src1w = jnp.pad(A1[0].reshape(NW, EPW1), pad)
    dst1w = jnp.pad(A1[1].reshape(NW, EPW1), pad,
                    constant_values=N1)
    src0t = A0[0].reshape(NW, CH, CW)
    dst0t = A0[1].reshape(NW, CH, CW)

    deg0p, deg1p = _sc_degrees(dst0w, dst1w)

    dis0, dis1, s1c, w1, vl = pl.pallas_call(
        _tc_a_body,
        out_shape=[_f32(1, N2), _f32(1, N1P), _f32(N1, 1), _f32(1, 128),
                   _f32(1, 128)],
    )(deg0p, deg1p, z, Wd, bd.reshape(1, -1), delta, U1, Wg0)

    s1p = jnp.pad(s1c.reshape(N1), (0, N1P - N1))
    t1p = _sc_a1_agg(s1p, dis1.reshape(N1P), src1w, dst1w)

    dis0c = dis0.reshape(N2, 1)

    q128 = pl.pallas_call(
        _tc_bg_body,
        grid=(_NRB,),
        in_specs=[_full_spec((N1, NW)), _full_spec((N1, 1)),
                  _full_spec((N1, 1)), _full_spec((1, 128)),
                  _full_spec((1, 128)), _full_spec((128, 128)),
                  pl.BlockSpec((_RB, N1), lambda i: (i, 0)), _row_spec(1)],
        out_specs=_row_spec(128),
        out_shape=_f32(N2, 128),
    )(t1p[:, :N1].T, dis1[:, :N1].reshape(N1, 1), s1c, w1,
      bg0.reshape(1, -1), Wg1, U0, dis0c)

    xl0 = pl.pallas_call(
        _tc_wl_body,
        grid=(10,),
        in_specs=[_full_spec((1, 128)),
                  pl.BlockSpec((128, 16000), lambda i: (0, i)),
                  pl.BlockSpec((1, 16000), lambda i: (0, i))],
        out_specs=pl.BlockSpec((1, 16000), lambda i: (0, i)),
        out_shape=_f32(1, 16 * N2),
    )(vl, Wl, bl.reshape(1, -1)).reshape(N2, 16)

    q64, y1p = pl.pallas_call(
        _tc_bl_body,
        grid=(_NRB,),
        in_specs=[_row_spec(16), _row_spec(1),
                  _full_spec((16, 64)), _full_spec((16, 64)),
                  _full_spec((1, 64))],
        out_specs=[_row_spec(64), _row_spec(64)],
        out_shape=[_f32(N2, 64), _f32(N2, 64)],
    )(xl0, dis0c, Wc0, Wp0, bp0.reshape(1, -1))

    zeros128 = jnp.zeros((RPT, 128), f32)
    qb = jnp.pad(q64, ((0, 0), (0, 64)))
    tpa = _sc_agg128(q128, src0t, dst0t, zeros128)
    tpb = _sc_agg128(qb, src0t, dst0t, zeros128)

    y1, q2 = pl.pallas_call(
        _tc_c_body,
        grid=(_NRB,),
        in_specs=[pl.BlockSpec((NC, _RB, 128), lambda i: (0, i, 0)),
                  _row_spec(64), _row_spec(1), _row_spec(64),
                  _full_spec((1, 64)), _full_spec((64, 128))],
        out_specs=[_row_spec(64), _row_spec(128)],
        out_shape=[_f32(N2, 64), _f32(N2, 128)],
    )(tpb, q64, dis0c, y1p, bc0.reshape(1, -1), Wc1)

    t2p = _sc_agg128(q2, src0t, dst0t, zeros128)

    out = pl.pallas_call(
        _tc_d_body,
        grid=(_NRB,),
        in_specs=[pl.BlockSpec((NC, _RB, 128), lambda i: (0, i, 0)),
                  _row_spec(128), _row_spec(1), _row_spec(64),
                  pl.BlockSpec((NC, _RB, 128), lambda i: (0, i, 0)),
                  _row_spec(128), _full_spec((1, 128)),
                  _full_spec((1, 128)), _full_spec((64, 128)),
                  _full_spec((1, 128)), _full_spec((128, 64)),
                  _full_spec((128, 64)), _full_spec((1, 64)),
                  _full_spec((64, 2)), _full_spec((1, 2))],
        out_specs=_row_spec(128),
        out_shape=_f32(N2, 128),
    )(t2p, q2, dis0c, y1, tpa, q128, bg1.reshape(1, -1),
      bc1.reshape(1, -1), Wp1, bp1.reshape(1, -1),
      Wa0[:128], Wa0[128:], ba0.reshape(1, -1), Wa1, ba1.reshape(1, -1))

    return out


# speedup vs baseline: 1.4150x; 1.0877x over previous
import functools

import jax
import jax.numpy as jnp
from jax import lax
from jax.experimental import pallas as pl
from jax.experimental.pallas import tpu as pltpu
from jax.experimental.pallas import tpu_sc as plsc

N2, N1 = 10000, 1000
E2, E1 = 320000, 16000
NC, NS, L = 2, 16, 16
NW = NC * NS
EPW0 = E2 // NW
EPW1 = E1 // NW
EPW1P = 512
N1P = 1024
CW = 80
CH = 125
N2P = 10112
RPT = N2P // NS

_MESH = plsc.VectorSubcoreMesh(core_axis_name="c", subcore_axis_name="s")
_SC_PARAMS = pltpu.CompilerParams(needs_layout_passes=False)


def _f32(*shape):
    return jax.ShapeDtypeStruct(shape, jnp.float32)


@functools.partial(
    pl.kernel,
    out_type=(_f32(NW, N2), _f32(NW, N1P)),
    mesh=_MESH,
    compiler_params=_SC_PARAMS,
    scratch_types=[
        pltpu.VMEM((EPW0,), jnp.int32),
        pltpu.VMEM((N2,), jnp.float32),
        pltpu.VMEM((EPW1P,), jnp.int32),
        pltpu.VMEM((1024,), jnp.float32),
    ],
)
def _sc_degrees(dst0_hbm, dst1_hbm, deg0p, deg1p, idx0_v, deg0_v, idx1_v, deg1_v):
    cid = lax.axis_index("c")
    sid = lax.axis_index("s")
    wid = cid * NS + sid
    ones = jnp.ones((L,), jnp.float32)
    zeros = jnp.zeros((L,), jnp.float32)

    pltpu.sync_copy(dst0_hbm.at[wid], idx0_v)

    def zero0(i, _):
        deg0_v[pl.ds(i * L, L)] = zeros
        return 0

    lax.fori_loop(0, N2 // L, zero0, 0)

    def acc0(i, _):
        idx = idx0_v[pl.ds(i * L, L)]
        plsc.addupdate_scatter(deg0_v, [idx], ones)
        return 0

    lax.fori_loop(0, EPW0 // L, acc0, 0)
    pltpu.sync_copy(deg0_v, deg0p.at[wid])

    def zero1(i, _):
        deg1_v[pl.ds(i * L, L)] = zeros
        return 0

    lax.fori_loop(0, 1024 // L, zero1, 0)
    pltpu.sync_copy(dst1_hbm.at[wid], idx1_v)

    def acc1(i, _):
        idx = idx1_v[pl.ds(i * L, L)]
        plsc.addupdate_scatter(deg1_v, [idx], ones)
        return 0

    lax.fori_loop(0, EPW1P // L, acc1, 0)
    pltpu.sync_copy(deg1_v, deg1p.at[wid])


@functools.partial(
    pl.kernel,
    out_type=_f32(NW, N1P),
    mesh=_MESH,
    compiler_params=_SC_PARAMS,
    scratch_types=[
        pltpu.VMEM((N1P,), jnp.float32),
        pltpu.VMEM((N1P,), jnp.float32),
        pltpu.VMEM((EPW1P,), jnp.int32),
        pltpu.VMEM((EPW1P,), jnp.int32),
        pltpu.VMEM((N1P,), jnp.float32),
    ],
)
def _sc_a1_agg(s1_hbm, dis1_hbm, src1_hbm, dst1_hbm, t1p,
               s1_v, dis1_v, src_v, dst_v, t1_v):
    cid = lax.axis_index("c")
    sid = lax.axis_index("s")
    wid = cid * NS + sid

    pltpu.sync_copy(s1_hbm, s1_v)
    pltpu.sync_copy(dis1_hbm, dis1_v)

    def zero(i, _):
        t1_v[pl.ds(i * L, L)] = jnp.zeros((L,), jnp.float32)
        return 0

    lax.fori_loop(0, 1024 // L, zero, 0)

    pltpu.sync_copy(src1_hbm.at[wid], src_v)
    pltpu.sync_copy(dst1_hbm.at[wid], dst_v)

    def acc(i, _):
        sidx = src_v[pl.ds(i * L, L)]
        didx = dst_v[pl.ds(i * L, L)]
        vals = plsc.load_gather(s1_v, [sidx]) * plsc.load_gather(dis1_v, [sidx])
        plsc.addupdate_scatter(t1_v, [didx], vals)
        return 0

    lax.fori_loop(0, EPW1P // L, acc, 0)
    pltpu.sync_copy(t1_v, t1p.at[wid])


def _agg_pass(q_hbm, src_v, dst_v, rows_v, acc_sh, sems, nch):
    def step(j, _):
        pltpu.async_copy(q_hbm.at[src_v.at[j]], rows_v, sems).wait()
        pltpu.sync_copy(rows_v, acc_sh.at[dst_v.at[j]], add=True)
        return 0

    lax.fori_loop(0, nch, step, 0)


_AGG_SCRATCH = [
    pltpu.VMEM((CH, CW), jnp.int32),
    pltpu.VMEM((CH, CW), jnp.int32),
    pltpu.VMEM((CW, 128), jnp.float32),
    pltpu.VMEM_SHARED((N2P, 128), jnp.float32),
    pltpu.SemaphoreType.DMA,
]


@functools.partial(
    pl.kernel,
    out_type=_f32(NC, N2P, 128),
    mesh=_MESH,
    compiler_params=_SC_PARAMS,
    scratch_types=_AGG_SCRATCH,
)
def _sc_agg128(q_hbm, src_hbm, dst_hbm, zeros_hbm, outp,
               src_v, dst_v, rows_v, acc_sh, sems):
    cid = lax.axis_index("c")
    sid = lax.axis_index("s")
    wid = cid * NS + sid
    base = sid * RPT

    pltpu.sync_copy(src_hbm.at[wid], src_v)
    pltpu.sync_copy(dst_hbm.at[wid], dst_v)
    pltpu.sync_copy(zeros_hbm, acc_sh.at[pl.ds(base, RPT)])
    plsc.subcore_barrier()
    _agg_pass(q_hbm, src_v, dst_v, rows_v, acc_sh, sems, CH)
    plsc.subcore_barrier()
    pltpu.sync_copy(acc_sh.at[pl.ds(base, RPT)],
                    outp.at[cid].at[pl.ds(base, RPT)])


_CB = 8000


@functools.partial(
    pl.kernel,
    out_type=_f32(NC, 16, N2),
    mesh=_MESH,
    compiler_params=_SC_PARAMS,
    scratch_types=[
        pltpu.VMEM((N2,), jnp.float32),
        pltpu.VMEM((N2,), jnp.float32),
        pltpu.VMEM((_CB,), jnp.int32),
        pltpu.VMEM((_CB,), jnp.int32),
    ],
)
def _sc_agg16(qxt_hbm, src_hbm, dst_hbm, outp, qrow_v, acc_v, src_v, dst_v):
    cid = lax.axis_index("c")
    sid = lax.axis_index("s")
    eh = E2 // NC

    pltpu.sync_copy(qxt_hbm.at[sid], qrow_v)

    def zero(i, _):
        acc_v[pl.ds(i * L, L)] = jnp.zeros((L,), jnp.float32)
        return 0

    lax.fori_loop(0, N2 // L, zero, 0)

    def chunk(c, _):
        pltpu.sync_copy(src_hbm.at[cid].at[c], src_v)
        pltpu.sync_copy(dst_hbm.at[cid].at[c], dst_v)

        def grp(i, _):
            s = src_v[pl.ds(i * L, L)]
            d = dst_v[pl.ds(i * L, L)]
            plsc.addupdate_scatter(acc_v, [d], plsc.load_gather(qrow_v, [s]))
            return 0

        lax.fori_loop(0, _CB // L, grp, 0)
        return 0

    lax.fori_loop(0, eh // _CB, chunk, 0)
    pltpu.sync_copy(acc_v, outp.at[cid].at[sid])


def _tc_a_body(deg0p, deg1p, z, wd, bd, delta, u1, wg0,
               dis0_o, dis1_o, s1_o, w1_o, vl_o):
    deg0 = jnp.sum(deg0p[...], axis=0, keepdims=True) + 1.0
    dis0_o[...] = lax.rsqrt(deg0)
    deg1 = jnp.sum(deg1p[...], axis=0, keepdims=True) + 1.0
    dis1_o[...] = lax.rsqrt(deg1)
    s1_o[...] = jnp.sum(u1[...], axis=1, keepdims=True)
    x = jnp.dot(z[...], wd[...], preferred_element_type=jnp.float32) \
        + bd[...] + delta[...]
    w1_o[...] = jnp.dot(x[:, :128], wg0[...], preferred_element_type=jnp.float32)
    vl_o[...] = x[:, 128:]


def _tc_bg_body(t1pt, dis1c, s1c, w1, bg0, wg1, u0, dis0c, q128_o):
    q1c = dis1c[...] * s1c[...]
    t1c = jnp.sum(t1pt[...], axis=1, keepdims=True)
    a1 = dis1c[...] * (t1c + q1c)
    pre = a1 * w1[...] + bg0[...]
    xg1 = jnp.where(pre >= 0, pre, 0.01 * pre)
    g = jnp.dot(xg1, wg1[...], preferred_element_type=jnp.float32)
    h2 = jnp.dot(u0[...], g, preferred_element_type=jnp.float32)
    q128_o[...] = h2 * dis0c[...]


def _tc_wl_body(vl, wl, bl, out_o):
    out_o[...] = jnp.dot(vl[...], wl[...], preferred_element_type=jnp.float32) \
        + bl[...]


def _tc_bl_body(xl0, dis0c, wp0, bp0, qx_o, y1p_o):
    qx_o[...] = xl0[...] * dis0c[...]
    y1p_o[...] = jnp.dot(xl0[...], wp0[...], preferred_element_type=jnp.float32) \
        + bp0[...]


def _tc_c_body(tpb, qx, dis0c, y1p, wc0, bc0, wc1, y1_o, q2_o):
    d = dis0c[...]
    z = tpb[0] + tpb[1] + qx[...]
    a = jnp.dot(z, wc0[...], preferred_element_type=jnp.float32)
    c0 = d * a + bc0[...]
    y1 = jnp.where(c0 >= 0, c0, 0.01 * c0) + y1p[...]
    y1_o[...] = y1
    q2_o[...] = jnp.dot(y1, wc1[...], preferred_element_type=jnp.float32) * d


def _tc_d_body(t2p, q2, dis0c, y1, tpa, q128, bg1, bc1, wp1, bp1, wa0g,
               wa0l, ba0, wa1, ba1, out_o):
    d = dis0c[...]
    t2 = t2p[0] + t2p[1]
    xl = d * (t2 + q2[...]) + bc1[...] \
        + jnp.dot(y1[...], wp1[...], preferred_element_type=jnp.float32) \
        + bp1[...]
    ta = tpa[0] + tpa[1]
    xgv = d * (ta + q128[...]) + bg1[...]
    h = jnp.dot(xgv, wa0g[...], preferred_element_type=jnp.float32) \
        + jnp.dot(xl, wa0l[...], preferred_element_type=jnp.float32) + ba0[...]
    h = jnp.maximum(h, 0.0)
    la = jnp.dot(h, wa1[...], preferred_element_type=jnp.float32) + ba1[...]
    d = la[:, 0:1] - la[:, 1:2]
    w0 = 1.0 / (1.0 + jnp.exp(-d))
    out_o[...] = w0 * xgv + (1.0 - w0) * xl


_RB = 1000
_NRB = N2 // _RB


def _row_spec(w):
    return pl.BlockSpec((_RB, w), lambda i: (i, 0))


def _full_spec(shape):
    return pl.BlockSpec(shape, lambda i: tuple(0 for _ in shape))


def kernel(z, Wd, bd, Wl, bl, Wg0, bg0, Wg1, bg1, Wc0, bc0, Wc1, bc1,
           Wp0, bp0, Wp1, bp1, Wa0, ba0, Wa1, ba1, U0, U1, A0, A1, batch_size):
    f32 = jnp.float32
    delta = (jnp.asarray(batch_size) - z.shape[0]).astype(f32).reshape(1, 1)

    dst0w = A0[1].reshape(NW, EPW0)
    pad = ((0, 0), (0, EPW1P - EPW1))
    src1w = jnp.pad(A1[0].reshape(NW, EPW1), pad)
    dst1w = jnp.pad(A1[1].reshape(NW, EPW1), pad,
                    constant_values=N1)
    src0t = A0[0].reshape(NW, CH, CW)
    dst0t = A0[1].reshape(NW, CH, CW)

    deg0p, deg1p = _sc_degrees(dst0w, dst1w)

    dis0, dis1, s1c, w1, vl = pl.pallas_call(
        _tc_a_body,
        out_shape=[_f32(1, N2), _f32(1, N1P), _f32(N1, 1), _f32(1, 128),
                   _f32(1, 128)],
    )(deg0p, deg1p, z, Wd, bd.reshape(1, -1), delta, U1, Wg0)

    s1p = jnp.pad(s1c.reshape(N1), (0, N1P - N1))
    t1p = _sc_a1_agg(s1p, dis1.reshape(N1P), src1w, dst1w)

    dis0c = dis0.reshape(N2, 1)

    q128 = pl.pallas_call(
        _tc_bg_body,
        grid=(_NRB,),
        in_specs=[_full_spec((N1, NW)), _full_spec((N1, 1)),
                  _full_spec((N1, 1)), _full_spec((1, 128)),
                  _full_spec((1, 128)), _full_spec((128, 128)),
                  pl.BlockSpec((_RB, N1), lambda i: (i, 0)), _row_spec(1)],
        out_specs=_row_spec(128),
        out_shape=_f32(N2, 128),
    )(t1p[:, :N1].T, dis1[:, :N1].reshape(N1, 1), s1c, w1,
      bg0.reshape(1, -1), Wg1, U0, dis0c)

    xl0 = pl.pallas_call(
        _tc_wl_body,
        grid=(10,),
        in_specs=[_full_spec((1, 128)),
                  pl.BlockSpec((128, 16000), lambda i: (0, i)),
                  pl.BlockSpec((1, 16000), lambda i: (0, i))],
        out_specs=pl.BlockSpec((1, 16000), lambda i: (0, i)),
        out_shape=_f32(1, 16 * N2),
    )(vl, Wl, bl.reshape(1, -1)).reshape(N2, 16)

    qx16, y1p = pl.pallas_call(
        _tc_bl_body,
        grid=(_NRB,),
        in_specs=[_row_spec(16), _row_spec(1),
                  _full_spec((16, 64)), _full_spec((1, 64))],
        out_specs=[_row_spec(16), _row_spec(64)],
        out_shape=[_f32(N2, 16), _f32(N2, 64)],
    )(xl0, dis0c, Wp0, bp0.reshape(1, -1))

    zeros128 = jnp.zeros((RPT, 128), f32)
    tpa = _sc_agg128(q128, src0t, dst0t, zeros128)
    src0e = A0[0].reshape(NC, E2 // NC // _CB, _CB)
    dst0e = A0[1].reshape(NC, E2 // NC // _CB, _CB)
    tpb = _sc_agg16(qx16.T, src0e, dst0e)

    y1, q2 = pl.pallas_call(
        _tc_c_body,
        grid=(_NRB,),
        in_specs=[pl.BlockSpec((NC, _RB, 16), lambda i: (0, i, 0)),
                  _row_spec(16), _row_spec(1), _row_spec(64),
                  _full_spec((16, 64)), _full_spec((1, 64)),
                  _full_spec((64, 128))],
        out_specs=[_row_spec(64), _row_spec(128)],
        out_shape=[_f32(N2, 64), _f32(N2, 128)],
    )(jnp.transpose(tpb, (0, 2, 1)), qx16, dis0c, y1p, Wc0,
      bc0.reshape(1, -1), Wc1)

    t2p = _sc_agg128(q2, src0t, dst0t, zeros128)

    out = pl.pallas_call(
        _tc_d_body,
        grid=(_NRB,),
        in_specs=[pl.BlockSpec((NC, _RB, 128), lambda i: (0, i, 0)),
                  _row_spec(128), _row_spec(1), _row_spec(64),
                  pl.BlockSpec((NC, _RB, 128), lambda i: (0, i, 0)),
                  _row_spec(128), _full_spec((1, 128)),
                  _full_spec((1, 128)), _full_spec((64, 128)),
                  _full_spec((1, 128)), _full_spec((128, 64)),
                  _full_spec((128, 64)), _full_spec((1, 64)),
                  _full_spec((64, 2)), _full_spec((1, 2))],
        out_specs=_row_spec(128),
        out_shape=_f32(N2, 128),
    )(t2p, q2, dis0c, y1, tpa, q128, bg1.reshape(1, -1),
      bc1.reshape(1, -1), Wp1, bp1.reshape(1, -1),
      Wa0[:128], Wa0[128:], ba0.reshape(1, -1), Wa1, ba1.reshape(1, -1))

    return out
